# baseline (speedup 1.0000x reference)
import functools

import jax
import jax.numpy as jnp
import numpy as np
from jax import lax
from jax.experimental import pallas as pl
from jax.experimental.pallas import tpu as pltpu
from jax.experimental.pallas import tpu_sc as plsc

B, C, T, H, W = 8, 256, 16, 28, 28
R = B * C * T
BLK = 128
_NVALID = (H - 1) * (W - 2)


def _tc_body(x_ref, o_ref):
    blk = x_ref[...]
    a = blk[:, 0:8, :] + blk[:, 8:16, :] + blk[:, 16:24, :]
    b4 = blk[:, 24:28, :]
    ii = lax.broadcasted_iota(jnp.int32, (BLK, 4, W), 1)
    sa = jnp.sum(a, axis=1)
    sb = jnp.sum(jnp.where(ii < 3, b4, 0.0), axis=1)
    y = sa + sb
    jj = lax.broadcasted_iota(jnp.int32, (BLK, W), 1)
    z = jnp.sum(jnp.where((jj >= 1) & (jj < W - 1), y, 0.0), axis=1)
    o_ref[...] = z * jnp.float32(1.0 / _NVALID)


@functools.cache
def _build_tc_pool():
    return pl.pallas_call(
        _tc_body,
        grid=(R // BLK,),
        in_specs=[pl.BlockSpec((BLK, H, W), lambda i: (i, 0, 0))],
        out_specs=pl.BlockSpec((BLK,), lambda i: (i,)),
        out_shape=jax.ShapeDtypeStruct((R,), jnp.float32),
    )


@jax.jit
def kernel(x):
    out = _build_tc_pool()(x.reshape(R, H, W))
    return out.reshape(B, C, T)


# speedup vs baseline: 11.0869x; 11.0869x over previous
import functools

import jax
import jax.numpy as jnp
import numpy as np
from jax import lax
from jax.experimental import pallas as pl
from jax.experimental.pallas import tpu as pltpu
from jax.experimental.pallas import tpu_sc as plsc

B, C, T, H, W = 8, 256, 16, 28, 28
_NVALID = (H - 1) * (W - 2)
ROWS = 4
STEPS = H // ROWS


def _tc_body(x_ref, o_ref):
    step = pl.program_id(1)
    blk = x_ref[...]
    s = jnp.sum(blk, axis=(1, 2))
    c = jnp.sum(blk[:, :, 0] + blk[:, :, W - 1], axis=1)
    c3 = jnp.sum(blk[:, ROWS - 1, 1:W - 1], axis=1)
    last = step == STEPS - 1
    ps = s - c - jnp.where(last, jnp.float32(1.0), jnp.float32(0.0)) * c3

    @pl.when(step == 0)
    def _():
        o_ref[...] = ps

    @pl.when((step > 0) & (~last))
    def _():
        o_ref[...] += ps

    @pl.when(last)
    def _():
        o_ref[...] = (o_ref[...] + ps) * jnp.float32(1.0 / _NVALID)


@functools.cache
def _build_tc_pool():
    return pl.pallas_call(
        _tc_body,
        grid=(B, STEPS),
        in_specs=[
            pl.BlockSpec((1, ROWS, W, T, C), lambda b, s: (b, s, 0, 0, 0))
        ],
        out_specs=pl.BlockSpec((1, T, C), lambda b, s: (b, 0, 0)),
        out_shape=jax.ShapeDtypeStruct((B, T, C), jnp.float32),
    )


@jax.jit
def kernel(x):
    xt = jnp.transpose(x, (0, 3, 4, 2, 1))
    out = _build_tc_pool()(xt)
    return jnp.transpose(out, (0, 2, 1))
